# baseline (speedup 1.0000x reference)
import functools

import jax
import jax.numpy as jnp
from jax import lax
from jax.experimental import pallas as pl
from jax.experimental.pallas import tpu as pltpu
from jax.experimental.pallas import tpu_sc as plsc

_NUM_SEGMENTS = 64
_EPS = 1e-05
_BLOCK = 5000
_PREC = jax.lax.Precision.DEFAULT
_N = 50000


def _sc_counts_kernel(batch_hbm, counts_hbm, ids_v, cnts_v, sem):
    wid = lax.axis_index("s") * 2 + lax.axis_index("c")

    @pl.when(wid == 0)
    def _work():
        pltpu.sync_copy(batch_hbm, ids_v)
        n = ids_v.shape[0]
        one16 = jnp.full((16,), 1, jnp.int32)
        nm1_16 = jnp.full((16,), n - 1, jnp.int32)

        def lower_bound(tgt):
            def body(_, carry):
                lo, hi = carry
                mid = lo + lax.shift_right_logical(hi - lo, one16)
                vals = plsc.load_gather(ids_v, [jnp.minimum(mid, nm1_16)])
                active = lo < hi
                below = vals < tgt
                go_right = jnp.logical_and(active, below)
                go_left = jnp.logical_and(active, jnp.logical_not(below))
                lo2 = jnp.where(go_right, mid + one16, lo)
                hi2 = jnp.where(go_left, mid, hi)
                return lo2, hi2

            lo = jnp.zeros((16,), jnp.int32)
            hi = jnp.full((16,), n, jnp.int32)
            for it in range(17):
                lo, hi = body(it, (lo, hi))
            return lo

        zero16 = jnp.zeros((16,), jnp.int32)
        iota16 = lax.broadcasted_iota(jnp.int32, (16,), 0)
        for g in range(4):
            s_vec = iota16 + jnp.full((16,), g * 16, jnp.int32)
            lb_lo = lower_bound(s_vec)
            lb_hi = lower_bound(s_vec + one16)
            cnt = (lb_hi - lb_lo).astype(jnp.float32)
            plsc.store_scatter(cnts_v, [s_vec, zero16], cnt)
        pltpu.sync_copy(cnts_v, counts_hbm)


def _counts_on_sc(batch_i32):
    mesh = plsc.VectorSubcoreMesh(core_axis_name="c", subcore_axis_name="s")
    k = functools.partial(
        pl.kernel,
        mesh=mesh,
        compiler_params=pltpu.CompilerParams(needs_layout_passes=False),
        out_type=jax.ShapeDtypeStruct((_NUM_SEGMENTS, 128), jnp.float32),
        scratch_types=[
            pltpu.VMEM((_N,), jnp.int32),
            pltpu.VMEM((_NUM_SEGMENTS, 128), jnp.float32),
            pltpu.SemaphoreType.DMA,
        ],
    )(_sc_counts_kernel)
    return k(batch_i32)


def _fused_kernel(x_ref, b_ref, cnt_ref, scale_ref, w_ref, bias_ref, o_ref,
                  sums, sumsq, ms, rw):
    i = pl.program_id(0)
    n = pl.num_programs(0)
    nb = n // 2
    S = sums.shape[0]

    @pl.when(i == 0)
    def _init():
        sums[...] = jnp.zeros_like(sums)
        sumsq[...] = jnp.zeros_like(sumsq)

    ids = b_ref[0, 0, :]
    B = ids.shape[0]

    @pl.when(i < nb)
    def _stats():
        x = x_ref[...]
        onehot = (jax.lax.broadcasted_iota(jnp.int32, (S, B), 0)
                  == ids[None, :]).astype(jnp.float32)
        sums[...] += jax.lax.dot(onehot, x, precision=_PREC)
        sumsq[...] += jax.lax.dot(onehot, x * x, precision=_PREC)

    @pl.when(i == nb - 1)
    def _finalize():
        inv = 1.0 / jnp.maximum(cnt_ref[:, :1], 1.0)
        m = sums[...] * inv
        q = sumsq[...] * inv
        s = scale_ref[...]
        var = jnp.maximum(q - m * m * (s * (2.0 - s)), 0.0)
        rstd = jax.lax.rsqrt(var + _EPS)
        ms[...] = m * s
        rw[...] = w_ref[...] * rstd

    @pl.when(i >= nb)
    def _norm():
        x = x_ref[...]
        onehot = (ids[:, None]
                  == jax.lax.broadcasted_iota(jnp.int32, (B, S), 1)
                  ).astype(jnp.float32)
        m_row = jax.lax.dot(onehot, ms[...], precision=_PREC)
        r_row = jax.lax.dot(onehot, rw[...], precision=_PREC)
        o_ref[...] = (x - m_row) * r_row + bias_ref[...]


@functools.partial(jax.jit, static_argnames=())
def kernel(x, batch, weight, bias, mean_scale):
    n, d = x.shape
    S = _NUM_SEGMENTS
    nb = n // _BLOCK
    batch_i32 = batch.astype(jnp.int32)
    counts = _counts_on_sc(batch_i32)
    b3 = batch_i32.reshape(nb, 1, _BLOCK)
    scale2 = mean_scale.reshape(1, d)
    w2 = weight.reshape(1, d)
    bias2 = bias.reshape(1, d)

    out = pl.pallas_call(
        _fused_kernel,
        grid=(2 * nb,),
        in_specs=[
            pl.BlockSpec((_BLOCK, d), lambda i: (i % nb, 0)),
            pl.BlockSpec((1, 1, _BLOCK), lambda i: (i % nb, 0, 0)),
            pl.BlockSpec((S, 128), lambda i: (0, 0)),
            pl.BlockSpec((1, d), lambda i: (0, 0)),
            pl.BlockSpec((1, d), lambda i: (0, 0)),
            pl.BlockSpec((1, d), lambda i: (0, 0)),
        ],
        out_specs=pl.BlockSpec((_BLOCK, d),
                               lambda i: (jnp.where(i < nb, 0, i - nb), 0)),
        out_shape=jax.ShapeDtypeStruct((n, d), jnp.float32),
        scratch_shapes=[
            pltpu.VMEM((S, d), jnp.float32),
            pltpu.VMEM((S, d), jnp.float32),
            pltpu.VMEM((S, d), jnp.float32),
            pltpu.VMEM((S, d), jnp.float32),
        ],
    )(x, b3, counts, scale2, w2, bias2)
    return out


# speedup vs baseline: 1.2319x; 1.2319x over previous
import functools

import jax
import jax.numpy as jnp
from jax.experimental import pallas as pl
from jax.experimental.pallas import tpu as pltpu

_NUM_SEGMENTS = 64
_EPS = 1e-05
_BLOCK = 5000
_PREC = jax.lax.Precision.DEFAULT


def _fused_kernel(x_ref, b_ref, scale_ref, w_ref, bias_ref, o_ref,
                  sums, sumsq, counts, ms, rw):
    i = pl.program_id(0)
    n = pl.num_programs(0)
    nb = n // 2
    S = sums.shape[0]

    @pl.when(i == 0)
    def _init():
        sums[...] = jnp.zeros_like(sums)
        sumsq[...] = jnp.zeros_like(sumsq)
        counts[...] = jnp.zeros_like(counts)

    ids = b_ref[0, 0, :]
    B = ids.shape[0]

    @pl.when(i < nb)
    def _stats():
        x = x_ref[...]
        onehot = (jax.lax.broadcasted_iota(jnp.int32, (S, B), 0)
                  == ids[None, :]).astype(jnp.float32)
        sums[...] += jax.lax.dot(onehot, x, precision=_PREC)
        sumsq[...] += jax.lax.dot(onehot, x * x, precision=_PREC)
        cnt = jnp.sum(onehot, axis=1, keepdims=True)
        counts[...] += jnp.broadcast_to(cnt, counts.shape)

    @pl.when(i == nb - 1)
    def _finalize():
        inv = 1.0 / jnp.maximum(counts[:, :1], 1.0)
        m = sums[...] * inv
        q = sumsq[...] * inv
        s = scale_ref[...]
        var = jnp.maximum(q - m * m * (s * (2.0 - s)), 0.0)
        rstd = jax.lax.rsqrt(var + _EPS)
        ms[...] = m * s
        rw[...] = w_ref[...] * rstd

    @pl.when(i >= nb)
    def _norm():
        x = x_ref[...]
        onehot = (ids[:, None]
                  == jax.lax.broadcasted_iota(jnp.int32, (B, S), 1)
                  ).astype(jnp.float32)
        m_row = jax.lax.dot(onehot, ms[...], precision=_PREC)
        r_row = jax.lax.dot(onehot, rw[...], precision=_PREC)
        o_ref[...] = (x - m_row) * r_row + bias_ref[...]


@functools.partial(jax.jit, static_argnames=())
def kernel(x, batch, weight, bias, mean_scale):
    n, d = x.shape
    S = _NUM_SEGMENTS
    nb = n // _BLOCK
    b3 = batch.astype(jnp.int32).reshape(nb, 1, _BLOCK)
    scale2 = mean_scale.reshape(1, d)
    w2 = weight.reshape(1, d)
    bias2 = bias.reshape(1, d)

    out = pl.pallas_call(
        _fused_kernel,
        grid=(2 * nb,),
        in_specs=[
            pl.BlockSpec((_BLOCK, d), lambda i: (i % nb, 0)),
            pl.BlockSpec((1, 1, _BLOCK), lambda i: (i % nb, 0, 0)),
            pl.BlockSpec((1, d), lambda i: (0, 0)),
            pl.BlockSpec((1, d), lambda i: (0, 0)),
            pl.BlockSpec((1, d), lambda i: (0, 0)),
        ],
        out_specs=pl.BlockSpec((_BLOCK, d),
                               lambda i: (jnp.where(i < nb, 0, i - nb), 0)),
        out_shape=jax.ShapeDtypeStruct((n, d), jnp.float32),
        scratch_shapes=[
            pltpu.VMEM((S, d), jnp.float32),
            pltpu.VMEM((S, d), jnp.float32),
            pltpu.VMEM((S, 128), jnp.float32),
            pltpu.VMEM((S, d), jnp.float32),
            pltpu.VMEM((S, d), jnp.float32),
        ],
    )(x, b3, scale2, w2, bias2)
    return out


# speedup vs baseline: 1.2653x; 1.0271x over previous
import functools

import jax
import jax.numpy as jnp
from jax.experimental import pallas as pl
from jax.experimental.pallas import tpu as pltpu

_NUM_SEGMENTS = 64
_EPS = 1e-05
_BLOCK = 2000
_CACHE_BLOCKS = 22
_PREC = jax.lax.Precision.DEFAULT


def _fused_kernel(x_ref, b_ref, scale_ref, w_ref, bias_ref, o_ref,
                  sums, sumsq, counts, ms, rw, xb):
    i = pl.program_id(0)
    n = pl.num_programs(0)
    nb = n // 2
    S = sums.shape[0]

    @pl.when(i == 0)
    def _init():
        sums[...] = jnp.zeros_like(sums)
        sumsq[...] = jnp.zeros_like(sumsq)
        counts[...] = jnp.zeros_like(counts)

    ids = b_ref[0, 0, :]
    B = ids.shape[0]

    @pl.when(i < nb)
    def _stats():
        x = x_ref[...]

        @pl.when(i < _CACHE_BLOCKS)
        def _retain():
            xb[pl.ds(i * B, B), :] = x.astype(jnp.bfloat16)
        onehot = (jax.lax.broadcasted_iota(jnp.int32, (S, B), 0)
                  == ids[None, :]).astype(jnp.float32)
        sums[...] += jax.lax.dot(onehot, x, precision=_PREC)
        sumsq[...] += jax.lax.dot(onehot, x * x, precision=_PREC)
        cnt = jnp.sum(onehot, axis=1, keepdims=True)
        counts[...] += jnp.broadcast_to(cnt, counts.shape)

    @pl.when(i == nb - 1)
    def _finalize():
        inv = 1.0 / jnp.maximum(counts[:, :1], 1.0)
        m = sums[...] * inv
        q = sumsq[...] * inv
        s = scale_ref[...]
        var = jnp.maximum(q - m * m * (s * (2.0 - s)), 0.0)
        rstd = jax.lax.rsqrt(var + _EPS)
        ms[...] = m * s
        rw[...] = w_ref[...] * rstd

    def _normalize(x):
        onehot = (ids[:, None]
                  == jax.lax.broadcasted_iota(jnp.int32, (B, S), 1)
                  ).astype(jnp.float32)
        m_row = jax.lax.dot(onehot, ms[...], precision=_PREC)
        r_row = jax.lax.dot(onehot, rw[...], precision=_PREC)
        o_ref[...] = (x - m_row) * r_row + bias_ref[...]

    @pl.when(jnp.logical_and(i >= nb, i - nb < _CACHE_BLOCKS))
    def _norm_cached():
        _normalize(xb[pl.ds((i - nb) * B, B), :].astype(jnp.float32))

    @pl.when(i - nb >= _CACHE_BLOCKS)
    def _norm_streamed():
        _normalize(x_ref[...])


@functools.partial(jax.jit, static_argnames=())
def kernel(x, batch, weight, bias, mean_scale):
    n, d = x.shape
    S = _NUM_SEGMENTS
    nb = n // _BLOCK
    b3 = batch.astype(jnp.int32).reshape(nb, 1, _BLOCK)
    scale2 = mean_scale.reshape(1, d)
    w2 = weight.reshape(1, d)
    bias2 = bias.reshape(1, d)

    out = pl.pallas_call(
        _fused_kernel,
        grid=(2 * nb,),
        in_specs=[
            pl.BlockSpec((_BLOCK, d),
                         lambda i: (jnp.where(
                             i < nb, i,
                             jnp.where(i - nb < _CACHE_BLOCKS,
                                       nb - 1, i - nb)), 0)),
            pl.BlockSpec((1, 1, _BLOCK), lambda i: (i % nb, 0, 0)),
            pl.BlockSpec((1, d), lambda i: (0, 0)),
            pl.BlockSpec((1, d), lambda i: (0, 0)),
            pl.BlockSpec((1, d), lambda i: (0, 0)),
        ],
        out_specs=pl.BlockSpec((_BLOCK, d),
                               lambda i: (jnp.where(i < nb, 0, i - nb), 0)),
        out_shape=jax.ShapeDtypeStruct((n, d), jnp.float32),
        scratch_shapes=[
            pltpu.VMEM((S, d), jnp.float32),
            pltpu.VMEM((S, d), jnp.float32),
            pltpu.VMEM((S, 128), jnp.float32),
            pltpu.VMEM((S, d), jnp.float32),
            pltpu.VMEM((S, d), jnp.float32),
            pltpu.VMEM((_CACHE_BLOCKS * _BLOCK, d), jnp.bfloat16),
        ],
        compiler_params=pltpu.CompilerParams(
            vmem_limit_bytes=110 * 1024 * 1024),
    )(x, b3, scale2, w2, bias2)
    return out
